# Initial kernel scaffold; baseline (speedup 1.0000x reference)
import jax
import jax.numpy as jnp
from jax.experimental import pallas as pl


def kernel(hidden_states, gate_w, exp_W1, exp_b1, exp_W2, exp_b2, sh_W1, sh_b1, sh_W2, sh_b2):
    raise NotImplementedError("write your pallas kernel here")


# speedup vs baseline: 2.6540x; 2.6540x over previous
import functools

import jax
import jax.numpy as jnp
from jax import lax
from jax.experimental import pallas as pl
from jax.experimental.pallas import tpu as pltpu
from jax.experimental.pallas import tpu_sc as plsc

DIM = 1024
NUM_EXPERTS = 8
TOP_K = 2
INNER = 4096
N_TOKENS = 4096
LANES = 128

BT = 256
NB = 40
R = NB * BT
NW = 32
ROWS_PER_W = R // NW
CHUNK = 64
NCH = ROWS_PER_W // CHUNK
ZROWS = TOP_K * N_TOKENS + NW

_SQRT_HALF = 0.7071067811865476


def _gelu_exact(x):
    return 0.5 * x * (1.0 + lax.erf(x * _SQRT_HALF))


def _gate_body(x_ref, gw_ref, w_ref, i_ref):
    x = x_ref[...]
    gw = gw_ref[...]
    logits = lax.dot_general(
        x, gw, (((1,), (1,)), ((), ())),
        preferred_element_type=jnp.float32)
    col = lax.broadcasted_iota(jnp.int32, logits.shape, 1)
    valid = col < NUM_EXPERTS
    logits = jnp.where(valid, logits, jnp.float32(-1e30))
    m = jnp.max(logits, axis=1, keepdims=True)
    ex = jnp.where(valid, jnp.exp(logits - m), 0.0)
    probs = ex / jnp.sum(ex, axis=1, keepdims=True)
    p1 = jnp.max(probs, axis=1, keepdims=True)
    i1 = jnp.min(jnp.where(probs == p1, col, NUM_EXPERTS), axis=1,
                 keepdims=True)
    probs2 = jnp.where(col == i1, jnp.float32(-1.0), probs)
    p2 = jnp.max(probs2, axis=1, keepdims=True)
    i2 = jnp.min(jnp.where(probs2 == p2, col, NUM_EXPERTS), axis=1,
                 keepdims=True)
    w_ref[...] = jnp.where(col == 0, p1, jnp.where(col == 1, p2, 0.0))
    i_ref[...] = jnp.where(col == 0, i1, jnp.where(col == 1, i2, 0))


def _gate(x, gate_w):
    gw_pad = jnp.zeros((LANES, DIM), jnp.float32).at[:NUM_EXPERTS].set(gate_w)
    w_pad, i_pad = pl.pallas_call(
        _gate_body,
        grid=(N_TOKENS // 512,),
        in_specs=[
            pl.BlockSpec((512, DIM), lambda b: (b, 0)),
            pl.BlockSpec((LANES, DIM), lambda b: (0, 0)),
        ],
        out_specs=[
            pl.BlockSpec((512, LANES), lambda b: (b, 0)),
            pl.BlockSpec((512, LANES), lambda b: (b, 0)),
        ],
        out_shape=[
            jax.ShapeDtypeStruct((N_TOKENS, LANES), jnp.float32),
            jax.ShapeDtypeStruct((N_TOKENS, LANES), jnp.int32),
        ],
    )(x, gw_pad)
    return w_pad[:, :TOP_K], i_pad[:, :TOP_K]


def _route_metadata(topk_w, topk_idx):
    ar = jnp.arange(N_TOKENS, dtype=jnp.int32)
    e_all = jnp.concatenate([topk_idx[:, 0], topk_idx[:, 1]])
    w_all = jnp.concatenate([topk_w[:, 0], topk_w[:, 1]])
    tok_all = jnp.concatenate([ar, ar])
    dest_all = jnp.concatenate([ar, ar + N_TOKENS])
    oh = (e_all[:, None] == jnp.arange(NUM_EXPERTS, dtype=jnp.int32)[None, :]
          ).astype(jnp.int32)
    csum = jnp.cumsum(oh, axis=0)
    rank = jnp.take_along_axis(csum, e_all[:, None], axis=1)[:, 0] - 1
    counts = csum[-1]
    pc = ((counts + BT - 1) // BT) * BT
    ends = jnp.cumsum(pc)
    offs = ends - pc
    pos = offs[e_all] + rank
    slot = jnp.arange(R, dtype=jnp.int32)
    sorted_tok = jnp.zeros((R,), jnp.int32).at[pos].set(tok_all)
    sorted_w = jnp.zeros((R,), jnp.float32).at[pos].set(w_all)
    dump = TOP_K * N_TOKENS + slot // ROWS_PER_W
    sorted_dest = dump.astype(jnp.int32).at[pos].set(dest_all)
    block_expert = jnp.clip(
        jnp.searchsorted(ends, jnp.arange(NB, dtype=jnp.int32) * BT,
                         side="right"),
        0, NUM_EXPERTS - 1).astype(jnp.int32)
    return sorted_tok, sorted_w, sorted_dest, block_expert


def _sc_mesh():
    return plsc.VectorSubcoreMesh(core_axis_name="c", subcore_axis_name="s")


def _gather_body(x_hbm, idx_hbm, out_hbm, idx_v, rows_v, sem):
    wid = lax.axis_index("s") * 2 + lax.axis_index("c")
    base = wid * ROWS_PER_W

    def body(c, _):
        off = base + c * CHUNK
        pltpu.sync_copy(idx_hbm.at[pl.ds(off, CHUNK)], idx_v)
        pltpu.async_copy(x_hbm.at[idx_v], rows_v, sem).wait()
        pltpu.sync_copy(rows_v, out_hbm.at[pl.ds(off, CHUNK)])
        return 0

    lax.fori_loop(0, NCH, body, 0)


def _sc_gather(x, sorted_tok):
    run = pl.kernel(
        _gather_body,
        out_type=jax.ShapeDtypeStruct((R, DIM), jnp.float32),
        mesh=_sc_mesh(),
        scratch_types=[
            pltpu.VMEM((CHUNK,), jnp.int32),
            pltpu.VMEM((CHUNK, DIM), jnp.float32),
            pltpu.SemaphoreType.DMA,
        ],
    )
    return run(x, sorted_tok)


def _scatter_body(y_hbm, dest_hbm, z_hbm, idx_v, rows_v, sem):
    wid = lax.axis_index("s") * 2 + lax.axis_index("c")
    base = wid * ROWS_PER_W

    def body(c, _):
        off = base + c * CHUNK
        pltpu.sync_copy(dest_hbm.at[pl.ds(off, CHUNK)], idx_v)
        pltpu.sync_copy(y_hbm.at[pl.ds(off, CHUNK)], rows_v)
        pltpu.async_copy(rows_v, z_hbm.at[idx_v], sem).wait()
        return 0

    lax.fori_loop(0, NCH, body, 0)


def _sc_scatter(y_sorted, sorted_dest):
    run = pl.kernel(
        _scatter_body,
        out_type=jax.ShapeDtypeStruct((ZROWS, DIM), jnp.float32),
        mesh=_sc_mesh(),
        scratch_types=[
            pltpu.VMEM((CHUNK,), jnp.int32),
            pltpu.VMEM((CHUNK, DIM), jnp.float32),
            pltpu.SemaphoreType.DMA,
        ],
    )
    return run(y_sorted, sorted_dest)


def _group_ffn_body(be_ref, xs_ref, w1_ref, b1_ref, w2_ref, b2_ref, sw_ref,
                    out_ref):
    del be_ref
    x = xs_ref[...].astype(jnp.bfloat16)
    h = lax.dot_general(
        x, w1_ref[0], (((1,), (1,)), ((), ())),
        preferred_element_type=jnp.float32)
    h = _gelu_exact(h + b1_ref[0, 0])
    y = lax.dot_general(
        h.astype(jnp.bfloat16), w2_ref[0], (((1,), (1,)), ((), ())),
        preferred_element_type=jnp.float32)
    y = y + b2_ref[0, 0]
    out_ref[...] = y * sw_ref[0, 0][:, None]


def _group_ffn(x_sorted, sorted_w, block_expert, exp_W1, exp_b1, exp_W2,
               exp_b2):
    w1 = exp_W1.astype(jnp.bfloat16)
    w2 = exp_W2.astype(jnp.bfloat16)
    b1 = exp_b1.reshape(NUM_EXPERTS, 1, INNER)
    b2 = exp_b2.reshape(NUM_EXPERTS, 1, DIM)
    sw = sorted_w.reshape(NB, 1, BT)
    grid_spec = pltpu.PrefetchScalarGridSpec(
        num_scalar_prefetch=1,
        grid=(NB,),
        in_specs=[
            pl.BlockSpec((BT, DIM), lambda b, be: (b, 0)),
            pl.BlockSpec((1, INNER, DIM), lambda b, be: (be[b], 0, 0)),
            pl.BlockSpec((1, 1, INNER), lambda b, be: (be[b], 0, 0)),
            pl.BlockSpec((1, DIM, INNER), lambda b, be: (be[b], 0, 0)),
            pl.BlockSpec((1, 1, DIM), lambda b, be: (be[b], 0, 0)),
            pl.BlockSpec((1, 1, BT), lambda b, be: (b, 0, 0)),
        ],
        out_specs=pl.BlockSpec((BT, DIM), lambda b, be: (b, 0)),
    )
    return pl.pallas_call(
        _group_ffn_body,
        grid_spec=grid_spec,
        out_shape=jax.ShapeDtypeStruct((R, DIM), jnp.float32),
    )(block_expert, x_sorted, w1, b1, w2, b2, sw)


def _shared_body(x_ref, w1_ref, b1_ref, w2_ref, b2_ref, z0_ref, z1_ref,
                 out_ref):
    x = x_ref[...].astype(jnp.bfloat16)
    h = lax.dot_general(
        x, w1_ref[...], (((1,), (1,)), ((), ())),
        preferred_element_type=jnp.float32)
    h = _gelu_exact(h + b1_ref[0])
    y = lax.dot_general(
        h.astype(jnp.bfloat16), w2_ref[...], (((1,), (1,)), ((), ())),
        preferred_element_type=jnp.float32)
    out_ref[...] = y + b2_ref[0] + z0_ref[...] + z1_ref[...]


def _shared_combine(x, sh_W1, sh_b1, sh_W2, sh_b2, z):
    w1 = sh_W1.astype(jnp.bfloat16)
    w2 = sh_W2.astype(jnp.bfloat16)
    b1 = sh_b1.reshape(1, INNER)
    b2 = sh_b2.reshape(1, DIM)
    nblk = N_TOKENS // BT
    return pl.pallas_call(
        _shared_body,
        grid=(nblk,),
        in_specs=[
            pl.BlockSpec((BT, DIM), lambda b: (b, 0)),
            pl.BlockSpec((INNER, DIM), lambda b: (0, 0)),
            pl.BlockSpec((1, INNER), lambda b: (0, 0)),
            pl.BlockSpec((DIM, INNER), lambda b: (0, 0)),
            pl.BlockSpec((1, DIM), lambda b: (0, 0)),
            pl.BlockSpec((BT, DIM), lambda b: (b, 0)),
            pl.BlockSpec((BT, DIM), lambda b: (b + nblk, 0)),
        ],
        out_specs=pl.BlockSpec((BT, DIM), lambda b: (b, 0)),
        out_shape=jax.ShapeDtypeStruct((N_TOKENS, DIM), jnp.float32),
    )(x, w1, b1, w2, b2, z, z)


def kernel(hidden_states, gate_w, exp_W1, exp_b1, exp_W2, exp_b2, sh_W1,
           sh_b1, sh_W2, sh_b2):
    b, s, d = hidden_states.shape
    x = hidden_states.reshape(-1, d)
    topk_w, topk_idx = _gate(x, gate_w)
    sorted_tok, sorted_w, sorted_dest, block_expert = _route_metadata(
        topk_w, topk_idx)
    x_sorted = _sc_gather(x, sorted_tok)
    y_sorted = _group_ffn(x_sorted, sorted_w, block_expert, exp_W1, exp_b1,
                          exp_W2, exp_b2)
    z = _sc_scatter(y_sorted, sorted_dest)
    y = _shared_combine(x, sh_W1, sh_b1, sh_W2, sh_b2, z)
    return y.reshape(b, s, d)


# speedup vs baseline: 2.7206x; 1.0251x over previous
import jax
import jax.numpy as jnp
from jax import lax
from jax.experimental import pallas as pl
from jax.experimental.pallas import tpu as pltpu
from jax.experimental.pallas import tpu_sc as plsc

DIM = 1024
NUM_EXPERTS = 8
TOP_K = 2
INNER = 4096
N_TOKENS = 4096
LANES = 128
SL = DIM // LANES

BT = 256
NB = 40
R = NB * BT
NW = 32
ROWS_PER_W = R // NW
CHUNK = 32
NCH = ROWS_PER_W // CHUNK
ZROWS = TOP_K * N_TOKENS + NW

_SQRT_HALF = 0.7071067811865476


def _gelu_exact(x):
    return 0.5 * x * (1.0 + lax.erf(x * _SQRT_HALF))


def _gate_body(x_ref, gw_ref, w_ref, i_ref):
    x = x_ref[...]
    gw = gw_ref[...]
    logits = lax.dot_general(
        x, gw, (((1,), (1,)), ((), ())),
        preferred_element_type=jnp.float32)
    col = lax.broadcasted_iota(jnp.int32, logits.shape, 1)
    valid = col < NUM_EXPERTS
    logits = jnp.where(valid, logits, jnp.float32(-1e30))
    m = jnp.max(logits, axis=1, keepdims=True)
    ex = jnp.where(valid, jnp.exp(logits - m), 0.0)
    probs = ex / jnp.sum(ex, axis=1, keepdims=True)
    p1 = jnp.max(probs, axis=1, keepdims=True)
    i1 = jnp.min(jnp.where(probs == p1, col, NUM_EXPERTS), axis=1,
                 keepdims=True)
    probs2 = jnp.where(col == i1, jnp.float32(-1.0), probs)
    p2 = jnp.max(probs2, axis=1, keepdims=True)
    i2 = jnp.min(jnp.where(probs2 == p2, col, NUM_EXPERTS), axis=1,
                 keepdims=True)
    w_ref[...] = jnp.where(col == 0, p1, jnp.where(col == 1, p2, 0.0))
    i_ref[...] = jnp.where(col == 0, i1, jnp.where(col == 1, i2, 0))


def _gate(x, gate_w):
    gw_pad = jnp.zeros((LANES, DIM), jnp.float32).at[:NUM_EXPERTS].set(gate_w)
    w_pad, i_pad = pl.pallas_call(
        _gate_body,
        grid=(N_TOKENS // 512,),
        in_specs=[
            pl.BlockSpec((512, DIM), lambda b: (b, 0)),
            pl.BlockSpec((LANES, DIM), lambda b: (0, 0)),
        ],
        out_specs=[
            pl.BlockSpec((512, LANES), lambda b: (b, 0)),
            pl.BlockSpec((512, LANES), lambda b: (b, 0)),
        ],
        out_shape=[
            jax.ShapeDtypeStruct((N_TOKENS, LANES), jnp.float32),
            jax.ShapeDtypeStruct((N_TOKENS, LANES), jnp.int32),
        ],
    )(x, gw_pad)
    return w_pad[:, :TOP_K], i_pad[:, :TOP_K]


def _route_metadata(topk_w, topk_idx):
    ar = jnp.arange(N_TOKENS, dtype=jnp.int32)
    e_all = jnp.concatenate([topk_idx[:, 0], topk_idx[:, 1]])
    w_all = jnp.concatenate([topk_w[:, 0], topk_w[:, 1]])
    tok_all = jnp.concatenate([ar, ar])
    dest_all = jnp.concatenate([ar, ar + N_TOKENS])
    oh = (e_all[:, None] == jnp.arange(NUM_EXPERTS, dtype=jnp.int32)[None, :]
          ).astype(jnp.int32)
    csum = jnp.cumsum(oh, axis=0)
    rank = jnp.take_along_axis(csum, e_all[:, None], axis=1)[:, 0] - 1
    counts = csum[-1]
    nblocks = (counts + BT - 1) // BT
    pc = nblocks * BT
    ends = jnp.cumsum(pc)
    offs = ends - pc
    pos = offs[e_all] + rank
    slot = jnp.arange(R, dtype=jnp.int32)
    sorted_tok = jnp.zeros((R,), jnp.int32).at[pos].set(tok_all)
    sorted_w = jnp.zeros((R,), jnp.float32).at[pos].set(w_all)
    dump = TOP_K * N_TOKENS + slot // ROWS_PER_W
    sorted_dest = dump.astype(jnp.int32).at[pos].set(dest_all)
    bar = jnp.arange(NB, dtype=jnp.int32)
    nlive = jnp.sum(nblocks)
    live = (bar < nlive).astype(jnp.int32)
    last = jnp.maximum(nlive - 1, 0)
    be = jnp.clip(jnp.searchsorted(ends, bar * BT, side="right"),
                  0, NUM_EXPERTS - 1).astype(jnp.int32)
    be = jnp.where(live == 1, be, be[last])
    xb = jnp.where(live == 1, bar, last).astype(jnp.int32)
    return sorted_tok, sorted_w, sorted_dest, be, xb, live


def _sc_mesh():
    return plsc.VectorSubcoreMesh(core_axis_name="c", subcore_axis_name="s")


def _gather_body(x_hbm, idx_hbm, out_hbm, idx_v, b0, b1, b2, gsem, wsem):
    wid = lax.axis_index("s") * 2 + lax.axis_index("c")
    base = wid * ROWS_PER_W
    pltpu.sync_copy(idx_hbm.at[pl.ds(base, ROWS_PER_W)], idx_v)
    bufs = (b0, b1, b2)

    def start_gather(c):
        return pltpu.async_copy(
            x_hbm.at[idx_v.at[pl.ds(c * CHUNK, CHUNK)]], bufs[c % 3], gsem)

    gcps = [start_gather(0)]
    wcps = []
    for c in range(NCH):
        if c >= 2:
            wcps[c - 2].wait()
        if c + 1 < NCH:
            gcps.append(start_gather(c + 1))
        gcps[c].wait()
        wcps.append(pltpu.async_copy(
            bufs[c % 3], out_hbm.at[pl.ds(base + c * CHUNK, CHUNK)], wsem))
    wcps[NCH - 2].wait()
    wcps[NCH - 1].wait()


def _sc_gather(x, sorted_tok):
    run = pl.kernel(
        _gather_body,
        out_type=jax.ShapeDtypeStruct((R, DIM), jnp.float32),
        mesh=_sc_mesh(),
        scratch_types=[
            pltpu.VMEM((ROWS_PER_W,), jnp.int32),
            pltpu.VMEM((CHUNK, DIM), jnp.float32),
            pltpu.VMEM((CHUNK, DIM), jnp.float32),
            pltpu.VMEM((CHUNK, DIM), jnp.float32),
            pltpu.SemaphoreType.DMA,
            pltpu.SemaphoreType.DMA,
        ],
    )
    return run(x, sorted_tok)


def _scatter_body(y_hbm, dest_hbm, z_hbm, idx_v, b0, b1, b2, rsem, ssem):
    wid = lax.axis_index("s") * 2 + lax.axis_index("c")
    base = wid * ROWS_PER_W
    pltpu.sync_copy(dest_hbm.at[wid], idx_v)
    bufs = (b0, b1, b2)

    def start_read(c):
        return pltpu.async_copy(
            y_hbm.at[pl.ds(base + c * CHUNK, CHUNK)], bufs[c % 3], rsem)

    rcps = [start_read(0)]
    scps = []
    for c in range(NCH):
        if c >= 2:
            scps[c - 2].wait()
        if c + 1 < NCH:
            rcps.append(start_read(c + 1))
        rcps[c].wait()
        scps.append(pltpu.async_copy(
            bufs[c % 3], z_hbm.at[idx_v.at[c]], ssem))
    scps[NCH - 2].wait()
    scps[NCH - 1].wait()


def _sc_scatter(y_sorted, dest3):
    run = pl.kernel(
        _scatter_body,
        out_type=jax.ShapeDtypeStruct((ZROWS, DIM), jnp.float32),
        mesh=_sc_mesh(),
        scratch_types=[
            pltpu.VMEM((NCH, CHUNK), jnp.int32),
            pltpu.VMEM((CHUNK, DIM), jnp.float32),
            pltpu.VMEM((CHUNK, DIM), jnp.float32),
            pltpu.VMEM((CHUNK, DIM), jnp.float32),
            pltpu.SemaphoreType.DMA,
            pltpu.SemaphoreType.DMA,
        ],
    )
    return run(y_sorted, dest3)


def _group_ffn_body(be_ref, xb_ref, live_ref, xs_ref, w1_ref, b1_ref, w2_ref,
                    b2_ref, sw_ref, out_ref):
    del be_ref, xb_ref
    b = pl.program_id(0)

    @pl.when(live_ref[b] == 1)
    def _():
        x = xs_ref[...].astype(jnp.bfloat16)
        h = lax.dot_general(
            x, w1_ref[0], (((1,), (1,)), ((), ())),
            preferred_element_type=jnp.float32)
        h = _gelu_exact(h + b1_ref[0, 0])
        y = lax.dot_general(
            h.astype(jnp.bfloat16), w2_ref[0], (((1,), (1,)), ((), ())),
            preferred_element_type=jnp.float32)
        y = y + b2_ref[0, 0]
        out_ref[...] = y * sw_ref[0, 0][:, None]


def _group_ffn(x_sorted, sorted_w, be, xb, live, exp_W1, exp_b1, exp_W2,
               exp_b2):
    w1 = exp_W1.astype(jnp.bfloat16)
    w2 = exp_W2.astype(jnp.bfloat16)
    b1 = exp_b1.reshape(NUM_EXPERTS, 1, INNER)
    b2 = exp_b2.reshape(NUM_EXPERTS, 1, DIM)
    sw = sorted_w.reshape(NB, 1, BT)
    grid_spec = pltpu.PrefetchScalarGridSpec(
        num_scalar_prefetch=3,
        grid=(NB,),
        in_specs=[
            pl.BlockSpec((BT, DIM), lambda b, be, xb, lv: (xb[b], 0)),
            pl.BlockSpec((1, INNER, DIM), lambda b, be, xb, lv: (be[b], 0, 0)),
            pl.BlockSpec((1, 1, INNER), lambda b, be, xb, lv: (be[b], 0, 0)),
            pl.BlockSpec((1, DIM, INNER), lambda b, be, xb, lv: (be[b], 0, 0)),
            pl.BlockSpec((1, 1, DIM), lambda b, be, xb, lv: (be[b], 0, 0)),
            pl.BlockSpec((1, 1, BT), lambda b, be, xb, lv: (xb[b], 0, 0)),
        ],
        out_specs=pl.BlockSpec((BT, DIM), lambda b, be, xb, lv: (xb[b], 0)),
    )
    return pl.pallas_call(
        _group_ffn_body,
        grid_spec=grid_spec,
        out_shape=jax.ShapeDtypeStruct((R, DIM), jnp.float32),
    )(be, xb, live, x_sorted, w1, b1, w2, b2, sw)


def _shared_body(x_ref, w1_ref, b1_ref, w2_ref, b2_ref, z0_ref, z1_ref,
                 out_ref):
    x = x_ref[...].astype(jnp.bfloat16)
    h = lax.dot_general(
        x, w1_ref[...], (((1,), (1,)), ((), ())),
        preferred_element_type=jnp.float32)
    h = _gelu_exact(h + b1_ref[0])
    y = lax.dot_general(
        h.astype(jnp.bfloat16), w2_ref[...], (((1,), (1,)), ((), ())),
        preferred_element_type=jnp.float32)
    out_ref[...] = y + b2_ref[0] + z0_ref[...] + z1_ref[...]


def _shared_combine(x, sh_W1, sh_b1, sh_W2, sh_b2, z):
    w1 = sh_W1.astype(jnp.bfloat16)
    w2 = sh_W2.astype(jnp.bfloat16)
    b1 = sh_b1.reshape(1, INNER)
    b2 = sh_b2.reshape(1, DIM)
    nblk = N_TOKENS // BT
    return pl.pallas_call(
        _shared_body,
        grid=(nblk,),
        in_specs=[
            pl.BlockSpec((BT, DIM), lambda b: (b, 0)),
            pl.BlockSpec((INNER, DIM), lambda b: (0, 0)),
            pl.BlockSpec((1, INNER), lambda b: (0, 0)),
            pl.BlockSpec((DIM, INNER), lambda b: (0, 0)),
            pl.BlockSpec((1, DIM), lambda b: (0, 0)),
            pl.BlockSpec((BT, DIM), lambda b: (b, 0)),
            pl.BlockSpec((BT, DIM), lambda b: (b + nblk, 0)),
        ],
        out_specs=pl.BlockSpec((BT, DIM), lambda b: (b, 0)),
        out_shape=jax.ShapeDtypeStruct((N_TOKENS, DIM), jnp.float32),
    )(x, w1, b1, w2, b2, z, z)


def kernel(hidden_states, gate_w, exp_W1, exp_b1, exp_W2, exp_b2, sh_W1,
           sh_b1, sh_W2, sh_b2):
    b, s, d = hidden_states.shape
    x = hidden_states.reshape(-1, d)
    topk_w, topk_idx = _gate(x, gate_w)
    sorted_tok, sorted_w, sorted_dest, be, xb, live = _route_metadata(
        topk_w, topk_idx)
    x_sorted = _sc_gather(x, sorted_tok)
    y_sorted = _group_ffn(x_sorted, sorted_w, be, xb, live,
                          exp_W1, exp_b1, exp_W2, exp_b2)
    z = _sc_scatter(y_sorted, sorted_dest.reshape(NW, NCH, CHUNK))
    y = _shared_combine(x, sh_W1, sh_b1, sh_W2, sh_b2, z)
    return y.reshape(b, s, d)


# speedup vs baseline: 3.1996x; 1.1761x over previous
import jax
import jax.numpy as jnp
from jax import lax
from jax.experimental import pallas as pl
from jax.experimental.pallas import tpu as pltpu
from jax.experimental.pallas import tpu_sc as plsc

DIM = 1024
NUM_EXPERTS = 8
TOP_K = 2
INNER = 4096
N_TOKENS = 4096
LANES = 128
SL = DIM // LANES

BT = 256
NB = 40
R = NB * BT
NW = 32
ROWS_PER_W = R // NW
CHUNK = 32
NCH = ROWS_PER_W // CHUNK
ZROWS = TOP_K * N_TOKENS + NW

_SQRT_HALF = 0.7071067811865476


def _gelu_exact(x):
    return 0.5 * x * (1.0 + lax.erf(x * _SQRT_HALF))


def _gate_body(x_ref, gw_ref, w_ref, i_ref):
    x = x_ref[...]
    gw = gw_ref[...]
    logits = lax.dot_general(
        x, gw, (((1,), (1,)), ((), ())),
        preferred_element_type=jnp.float32)
    col = lax.broadcasted_iota(jnp.int32, logits.shape, 1)
    valid = col < NUM_EXPERTS
    logits = jnp.where(valid, logits, jnp.float32(-1e30))
    m = jnp.max(logits, axis=1, keepdims=True)
    ex = jnp.where(valid, jnp.exp(logits - m), 0.0)
    probs = ex / jnp.sum(ex, axis=1, keepdims=True)
    p1 = jnp.max(probs, axis=1, keepdims=True)
    i1 = jnp.min(jnp.where(probs == p1, col, NUM_EXPERTS), axis=1,
                 keepdims=True)
    probs2 = jnp.where(col == i1, jnp.float32(-1.0), probs)
    p2 = jnp.max(probs2, axis=1, keepdims=True)
    i2 = jnp.min(jnp.where(probs2 == p2, col, NUM_EXPERTS), axis=1,
                 keepdims=True)
    w_ref[...] = jnp.where(col == 0, p1, jnp.where(col == 1, p2, 0.0))
    i_ref[...] = jnp.where(col == 0, i1, jnp.where(col == 1, i2, 0))


def _gate(x, gate_w):
    gw_pad = jnp.zeros((LANES, DIM), jnp.float32).at[:NUM_EXPERTS].set(gate_w)
    w_pad, i_pad = pl.pallas_call(
        _gate_body,
        grid=(N_TOKENS // 512,),
        in_specs=[
            pl.BlockSpec((512, DIM), lambda b: (b, 0)),
            pl.BlockSpec((LANES, DIM), lambda b: (0, 0)),
        ],
        out_specs=[
            pl.BlockSpec((512, LANES), lambda b: (b, 0)),
            pl.BlockSpec((512, LANES), lambda b: (b, 0)),
        ],
        out_shape=[
            jax.ShapeDtypeStruct((N_TOKENS, LANES), jnp.float32),
            jax.ShapeDtypeStruct((N_TOKENS, LANES), jnp.int32),
        ],
    )(x, gw_pad)
    return w_pad[:, :TOP_K], i_pad[:, :TOP_K]


def _route_metadata(topk_w, topk_idx):
    ar = jnp.arange(N_TOKENS, dtype=jnp.int32)
    e_all = jnp.concatenate([topk_idx[:, 0], topk_idx[:, 1]])
    w_all = jnp.concatenate([topk_w[:, 0], topk_w[:, 1]])
    tok_all = jnp.concatenate([ar, ar])
    dest_all = jnp.concatenate([ar, ar + N_TOKENS])
    oh = (e_all[:, None] == jnp.arange(NUM_EXPERTS, dtype=jnp.int32)[None, :]
          ).astype(jnp.int32)
    csum = jnp.cumsum(oh, axis=0)
    rank = jnp.take_along_axis(csum, e_all[:, None], axis=1)[:, 0] - 1
    counts = csum[-1]
    nblocks = (counts + BT - 1) // BT
    pc = nblocks * BT
    ends = jnp.cumsum(pc)
    offs = ends - pc
    pos = offs[e_all] + rank
    slot = jnp.arange(R, dtype=jnp.int32)
    dump = (TOP_K * N_TOKENS + slot // ROWS_PER_W).astype(jnp.int32)
    init = jnp.stack(
        [jnp.zeros((R,), jnp.int32), dump, jnp.zeros((R,), jnp.int32)],
        axis=1)
    payload = jnp.stack(
        [tok_all, dest_all, lax.bitcast_convert_type(w_all, jnp.int32)],
        axis=1)
    smat = init.at[pos].set(payload)
    sorted_tok = smat[:, 0]
    sorted_dest = smat[:, 1]
    sorted_w = lax.bitcast_convert_type(smat[:, 2], jnp.float32)
    bar = jnp.arange(NB, dtype=jnp.int32)
    nlive = jnp.sum(nblocks)
    live = (bar < nlive).astype(jnp.int32)
    last = jnp.maximum(nlive - 1, 0)
    be = jnp.clip(jnp.sum((bar[:, None] * BT >= ends[None, :])
                          .astype(jnp.int32), axis=1),
                  0, NUM_EXPERTS - 1).astype(jnp.int32)
    be = jnp.where(live == 1, be, be[last])
    xb = jnp.where(live == 1, bar, last).astype(jnp.int32)
    return sorted_tok, sorted_w, sorted_dest, be, xb, live


def _sc_mesh():
    return plsc.VectorSubcoreMesh(core_axis_name="c", subcore_axis_name="s")


def _gather_body(x_hbm, idx_hbm, out_hbm, idx_v, b0, b1, b2, gsem, wsem):
    wid = lax.axis_index("s") * 2 + lax.axis_index("c")
    base = wid * ROWS_PER_W
    pltpu.sync_copy(idx_hbm.at[pl.ds(base, ROWS_PER_W)], idx_v)
    bufs = (b0, b1, b2)

    def start_gather(c):
        return pltpu.async_copy(
            x_hbm.at[idx_v.at[pl.ds(c * CHUNK, CHUNK)]], bufs[c % 3], gsem)

    gcps = [start_gather(0)]
    wcps = []
    for c in range(NCH):
        if c >= 2:
            wcps[c - 2].wait()
        if c + 1 < NCH:
            gcps.append(start_gather(c + 1))
        gcps[c].wait()
        wcps.append(pltpu.async_copy(
            bufs[c % 3], out_hbm.at[pl.ds(base + c * CHUNK, CHUNK)], wsem))
    wcps[NCH - 2].wait()
    wcps[NCH - 1].wait()


def _sc_gather(x, sorted_tok):
    run = pl.kernel(
        _gather_body,
        out_type=jax.ShapeDtypeStruct((R, DIM), jnp.float32),
        mesh=_sc_mesh(),
        scratch_types=[
            pltpu.VMEM((ROWS_PER_W,), jnp.int32),
            pltpu.VMEM((CHUNK, DIM), jnp.float32),
            pltpu.VMEM((CHUNK, DIM), jnp.float32),
            pltpu.VMEM((CHUNK, DIM), jnp.float32),
            pltpu.SemaphoreType.DMA,
            pltpu.SemaphoreType.DMA,
        ],
    )
    return run(x, sorted_tok)


def _scatter_body(y_hbm, dest_hbm, z_hbm, idx_v, b0, b1, b2, rsem, ssem):
    wid = lax.axis_index("s") * 2 + lax.axis_index("c")
    base = wid * ROWS_PER_W
    pltpu.sync_copy(dest_hbm.at[wid], idx_v)
    bufs = (b0, b1, b2)

    def start_read(c):
        return pltpu.async_copy(
            y_hbm.at[pl.ds(base + c * CHUNK, CHUNK)], bufs[c % 3], rsem)

    rcps = [start_read(0)]
    scps = []
    for c in range(NCH):
        if c >= 2:
            scps[c - 2].wait()
        if c + 1 < NCH:
            rcps.append(start_read(c + 1))
        rcps[c].wait()
        scps.append(pltpu.async_copy(
            bufs[c % 3], z_hbm.at[idx_v.at[c]], ssem))
    scps[NCH - 2].wait()
    scps[NCH - 1].wait()


def _sc_scatter(y_sorted, dest3):
    run = pl.kernel(
        _scatter_body,
        out_type=jax.ShapeDtypeStruct((ZROWS, DIM), jnp.float32),
        mesh=_sc_mesh(),
        scratch_types=[
            pltpu.VMEM((NCH, CHUNK), jnp.int32),
            pltpu.VMEM((CHUNK, DIM), jnp.float32),
            pltpu.VMEM((CHUNK, DIM), jnp.float32),
            pltpu.VMEM((CHUNK, DIM), jnp.float32),
            pltpu.SemaphoreType.DMA,
            pltpu.SemaphoreType.DMA,
        ],
    )
    return run(y_sorted, dest3)


def _group_ffn_body(be_ref, xb_ref, live_ref, xs_ref, w1_ref, b1_ref, w2_ref,
                    b2_ref, sw_ref, out_ref):
    del be_ref, xb_ref
    b = pl.program_id(0)

    @pl.when(live_ref[b] == 1)
    def _():
        x = xs_ref[...].astype(jnp.bfloat16)
        h = lax.dot_general(
            x, w1_ref[0].astype(jnp.bfloat16), (((1,), (1,)), ((), ())),
            preferred_element_type=jnp.float32)
        h = _gelu_exact(h + b1_ref[0, 0])
        y = lax.dot_general(
            h.astype(jnp.bfloat16), w2_ref[0], (((1,), (1,)), ((), ())),
            preferred_element_type=jnp.float32)
        y = y + b2_ref[0, 0]
        out_ref[...] = y * sw_ref[0, 0][:, None]


def _group_ffn(x_sorted, sorted_w, be, xb, live, exp_W1, exp_b1, exp_W2,
               exp_b2):
    w2 = exp_W2.astype(jnp.bfloat16)
    b1 = exp_b1.reshape(NUM_EXPERTS, 1, INNER)
    b2 = exp_b2.reshape(NUM_EXPERTS, 1, DIM)
    sw = sorted_w.reshape(NB, 1, BT)
    grid_spec = pltpu.PrefetchScalarGridSpec(
        num_scalar_prefetch=3,
        grid=(NB,),
        in_specs=[
            pl.BlockSpec((BT, DIM), lambda b, be, xb, lv: (xb[b], 0)),
            pl.BlockSpec((1, INNER, DIM), lambda b, be, xb, lv: (be[b], 0, 0)),
            pl.BlockSpec((1, 1, INNER), lambda b, be, xb, lv: (be[b], 0, 0)),
            pl.BlockSpec((1, DIM, INNER), lambda b, be, xb, lv: (be[b], 0, 0)),
            pl.BlockSpec((1, 1, DIM), lambda b, be, xb, lv: (be[b], 0, 0)),
            pl.BlockSpec((1, 1, BT), lambda b, be, xb, lv: (xb[b], 0, 0)),
        ],
        out_specs=pl.BlockSpec((BT, DIM), lambda b, be, xb, lv: (xb[b], 0)),
    )
    return pl.pallas_call(
        _group_ffn_body,
        grid_spec=grid_spec,
        out_shape=jax.ShapeDtypeStruct((R, DIM), jnp.float32),
    )(be, xb, live, x_sorted, exp_W1, b1, w2, b2, sw)


def _shared_body(x_ref, w1_ref, b1_ref, w2_ref, b2_ref, z0_ref, z1_ref,
                 out_ref):
    x = x_ref[...].astype(jnp.bfloat16)
    h = lax.dot_general(
        x, w1_ref[...].astype(jnp.bfloat16), (((1,), (1,)), ((), ())),
        preferred_element_type=jnp.float32)
    h = _gelu_exact(h + b1_ref[0])
    y = lax.dot_general(
        h.astype(jnp.bfloat16), w2_ref[...], (((1,), (1,)), ((), ())),
        preferred_element_type=jnp.float32)
    out_ref[...] = y + b2_ref[0] + z0_ref[...] + z1_ref[...]


def _shared_combine(x, sh_W1, sh_b1, sh_W2, sh_b2, z):
    w2 = sh_W2.astype(jnp.bfloat16)
    b1 = sh_b1.reshape(1, INNER)
    b2 = sh_b2.reshape(1, DIM)
    nblk = N_TOKENS // BT
    return pl.pallas_call(
        _shared_body,
        grid=(nblk,),
        in_specs=[
            pl.BlockSpec((BT, DIM), lambda b: (b, 0)),
            pl.BlockSpec((INNER, DIM), lambda b: (0, 0)),
            pl.BlockSpec((1, INNER), lambda b: (0, 0)),
            pl.BlockSpec((DIM, INNER), lambda b: (0, 0)),
            pl.BlockSpec((1, DIM), lambda b: (0, 0)),
            pl.BlockSpec((BT, DIM), lambda b: (b, 0)),
            pl.BlockSpec((BT, DIM), lambda b: (b + nblk, 0)),
        ],
        out_specs=pl.BlockSpec((BT, DIM), lambda b: (b, 0)),
        out_shape=jax.ShapeDtypeStruct((N_TOKENS, DIM), jnp.float32),
    )(x, sh_W1, b1, w2, b2, z, z)


def kernel(hidden_states, gate_w, exp_W1, exp_b1, exp_W2, exp_b2, sh_W1,
           sh_b1, sh_W2, sh_b2):
    b, s, d = hidden_states.shape
    x = hidden_states.reshape(-1, d)
    topk_w, topk_idx = _gate(x, gate_w)
    sorted_tok, sorted_w, sorted_dest, be, xb, live = _route_metadata(
        topk_w, topk_idx)
    x_sorted = _sc_gather(x, sorted_tok)
    y_sorted = _group_ffn(x_sorted, sorted_w, be, xb, live,
                          exp_W1, exp_b1, exp_W2, exp_b2)
    z = _sc_scatter(y_sorted, sorted_dest.reshape(NW, NCH, CHUNK))
    y = _shared_combine(x, sh_W1, sh_b1, sh_W2, sh_b2, z)
    return y.reshape(b, s, d)


# speedup vs baseline: 3.2169x; 1.0054x over previous
import jax
import jax.numpy as jnp
from jax import lax
from jax.experimental import pallas as pl
from jax.experimental.pallas import tpu as pltpu
from jax.experimental.pallas import tpu_sc as plsc

DIM = 1024
NUM_EXPERTS = 8
TOP_K = 2
INNER = 4096
N_TOKENS = 4096
LANES = 128
SL = DIM // LANES

BT = 256
NB = 40
R = NB * BT
NW = 32
ROWS_PER_W = R // NW
CHUNK = 32
NCH = ROWS_PER_W // CHUNK
ZROWS = TOP_K * N_TOKENS + NW

_SQRT_HALF = 0.7071067811865476


def _gelu_exact(x):
    return 0.5 * x * (1.0 + lax.erf(x * _SQRT_HALF))


def _gate_body(x_ref, gw_ref, w_ref, i_ref):
    x = x_ref[...]
    gw = gw_ref[...]
    logits = lax.dot_general(
        x, gw, (((1,), (1,)), ((), ())),
        preferred_element_type=jnp.float32)
    col = lax.broadcasted_iota(jnp.int32, logits.shape, 1)
    valid = col < NUM_EXPERTS
    logits = jnp.where(valid, logits, jnp.float32(-1e30))
    m = jnp.max(logits, axis=1, keepdims=True)
    ex = jnp.where(valid, jnp.exp(logits - m), 0.0)
    probs = ex / jnp.sum(ex, axis=1, keepdims=True)
    p1 = jnp.max(probs, axis=1, keepdims=True)
    i1 = jnp.min(jnp.where(probs == p1, col, NUM_EXPERTS), axis=1,
                 keepdims=True)
    probs2 = jnp.where(col == i1, jnp.float32(-1.0), probs)
    p2 = jnp.max(probs2, axis=1, keepdims=True)
    i2 = jnp.min(jnp.where(probs2 == p2, col, NUM_EXPERTS), axis=1,
                 keepdims=True)
    w_ref[...] = jnp.where(col == 0, p1, jnp.where(col == 1, p2, 0.0))
    i_ref[...] = jnp.where(col == 0, i1, jnp.where(col == 1, i2, 0))


def _gate(x, gate_w):
    gw_pad = jnp.zeros((LANES, DIM), jnp.float32).at[:NUM_EXPERTS].set(gate_w)
    w_pad, i_pad = pl.pallas_call(
        _gate_body,
        grid=(N_TOKENS // 512,),
        in_specs=[
            pl.BlockSpec((512, DIM), lambda b: (b, 0)),
            pl.BlockSpec((LANES, DIM), lambda b: (0, 0)),
        ],
        out_specs=[
            pl.BlockSpec((512, LANES), lambda b: (b, 0)),
            pl.BlockSpec((512, LANES), lambda b: (b, 0)),
        ],
        out_shape=[
            jax.ShapeDtypeStruct((N_TOKENS, LANES), jnp.float32),
            jax.ShapeDtypeStruct((N_TOKENS, LANES), jnp.int32),
        ],
    )(x, gw_pad)
    return w_pad[:, :TOP_K], i_pad[:, :TOP_K]


def _route_metadata(topk_w, topk_idx):
    ar = jnp.arange(N_TOKENS, dtype=jnp.int32)
    e_all = jnp.concatenate([topk_idx[:, 0], topk_idx[:, 1]])
    w_all = jnp.concatenate([topk_w[:, 0], topk_w[:, 1]])
    tok_all = jnp.concatenate([ar, ar])
    dest_all = jnp.concatenate([ar, ar + N_TOKENS])
    oh = (e_all[:, None] == jnp.arange(NUM_EXPERTS, dtype=jnp.int32)[None, :]
          ).astype(jnp.int32)
    csum = jnp.cumsum(oh, axis=0)
    rank = jnp.take_along_axis(csum, e_all[:, None], axis=1)[:, 0] - 1
    counts = csum[-1]
    nblocks = (counts + BT - 1) // BT
    pc = nblocks * BT
    ends = jnp.cumsum(pc)
    offs = ends - pc
    pos = offs[e_all] + rank
    slot = jnp.arange(R, dtype=jnp.int32)
    dump = (TOP_K * N_TOKENS + slot // ROWS_PER_W).astype(jnp.int32)
    init = jnp.stack(
        [jnp.zeros((R,), jnp.int32), dump, jnp.zeros((R,), jnp.int32)],
        axis=1)
    payload = jnp.stack(
        [tok_all, dest_all, lax.bitcast_convert_type(w_all, jnp.int32)],
        axis=1)
    smat = init.at[pos].set(payload)
    sorted_tok = smat[:, 0]
    sorted_dest = smat[:, 1]
    sorted_w = lax.bitcast_convert_type(smat[:, 2], jnp.float32)
    bar = jnp.arange(NB, dtype=jnp.int32)
    nlive = jnp.sum(nblocks)
    live = (bar < nlive).astype(jnp.int32)
    last = jnp.maximum(nlive - 1, 0)
    be = jnp.clip(jnp.sum((bar[:, None] * BT >= ends[None, :])
                          .astype(jnp.int32), axis=1),
                  0, NUM_EXPERTS - 1).astype(jnp.int32)
    be = jnp.where(live == 1, be, be[last])
    xb = jnp.where(live == 1, bar, last).astype(jnp.int32)
    return sorted_tok, sorted_w, sorted_dest, be, xb, live


def _sc_mesh():
    return plsc.VectorSubcoreMesh(core_axis_name="c", subcore_axis_name="s")


def _gather_body(x_hbm, idx_hbm, out_hbm, idx_v, b0, b1, b2, gsem, wsem):
    wid = lax.axis_index("s") * 2 + lax.axis_index("c")
    base = wid * ROWS_PER_W
    pltpu.sync_copy(idx_hbm.at[pl.ds(base, ROWS_PER_W)], idx_v)
    bufs = (b0, b1, b2)

    def start_gather(c):
        return pltpu.async_copy(
            x_hbm.at[idx_v.at[pl.ds(c * CHUNK, CHUNK)]], bufs[c % 3], gsem)

    gcps = [start_gather(0)]
    wcps = []
    for c in range(NCH):
        if c >= 2:
            wcps[c - 2].wait()
        if c + 1 < NCH:
            gcps.append(start_gather(c + 1))
        gcps[c].wait()
        wcps.append(pltpu.async_copy(
            bufs[c % 3], out_hbm.at[pl.ds(base + c * CHUNK, CHUNK)], wsem))
    wcps[NCH - 2].wait()
    wcps[NCH - 1].wait()


def _sc_gather(x, sorted_tok):
    run = pl.kernel(
        _gather_body,
        out_type=jax.ShapeDtypeStruct((R, DIM), jnp.float32),
        mesh=_sc_mesh(),
        scratch_types=[
            pltpu.VMEM((ROWS_PER_W,), jnp.int32),
            pltpu.VMEM((CHUNK, DIM), jnp.float32),
            pltpu.VMEM((CHUNK, DIM), jnp.float32),
            pltpu.VMEM((CHUNK, DIM), jnp.float32),
            pltpu.SemaphoreType.DMA,
            pltpu.SemaphoreType.DMA,
        ],
    )
    return run(x, sorted_tok)


def _scatter_body(y_hbm, dest_hbm, z_hbm, idx_v, b0, b1, b2, rsem, ssem):
    wid = lax.axis_index("s") * 2 + lax.axis_index("c")
    base = wid * ROWS_PER_W
    pltpu.sync_copy(dest_hbm.at[wid], idx_v)
    bufs = (b0, b1, b2)

    def start_read(c):
        return pltpu.async_copy(
            y_hbm.at[pl.ds(base + c * CHUNK, CHUNK)], bufs[c % 3], rsem)

    rcps = [start_read(0)]
    scps = []
    for c in range(NCH):
        if c >= 2:
            scps[c - 2].wait()
        if c + 1 < NCH:
            rcps.append(start_read(c + 1))
        rcps[c].wait()
        scps.append(pltpu.async_copy(
            bufs[c % 3], z_hbm.at[idx_v.at[c]], ssem))
    scps[NCH - 2].wait()
    scps[NCH - 1].wait()


def _sc_scatter(y_sorted, dest3):
    run = pl.kernel(
        _scatter_body,
        out_type=jax.ShapeDtypeStruct((ZROWS, DIM), jnp.float32),
        mesh=_sc_mesh(),
        scratch_types=[
            pltpu.VMEM((NCH, CHUNK), jnp.int32),
            pltpu.VMEM((CHUNK, DIM), jnp.float32),
            pltpu.VMEM((CHUNK, DIM), jnp.float32),
            pltpu.VMEM((CHUNK, DIM), jnp.float32),
            pltpu.SemaphoreType.DMA,
            pltpu.SemaphoreType.DMA,
        ],
    )
    return run(y_sorted, dest3)


def _ffn1_body(be_ref, xb_ref, live_ref, xs_ref, w1_ref, b1_ref, h_ref):
    del be_ref, xb_ref
    b = pl.program_id(0)

    @pl.when(live_ref[b] == 1)
    def _():
        x = xs_ref[...].astype(jnp.bfloat16)
        h = lax.dot_general(
            x, w1_ref[0].astype(jnp.bfloat16), (((1,), (1,)), ((), ())),
            preferred_element_type=jnp.float32)
        h_ref[...] = _gelu_exact(h + b1_ref[0, 0]).astype(jnp.bfloat16)


def _ffn2_body(be_ref, xb_ref, live_ref, h_ref, w2_ref, b2_ref, sw_ref,
               out_ref):
    del be_ref, xb_ref
    b = pl.program_id(0)

    @pl.when(live_ref[b] == 1)
    def _():
        y = lax.dot_general(
            h_ref[...], w2_ref[0].astype(jnp.bfloat16),
            (((1,), (1,)), ((), ())),
            preferred_element_type=jnp.float32)
        y = y + b2_ref[0, 0]
        out_ref[...] = y * sw_ref[0, 0][:, None]


def _group_ffn(x_sorted, sorted_w, be, xb, live, exp_W1, exp_b1, exp_W2,
               exp_b2):
    b1 = exp_b1.reshape(NUM_EXPERTS, 1, INNER)
    b2 = exp_b2.reshape(NUM_EXPERTS, 1, DIM)
    sw = sorted_w.reshape(NB, 1, BT)
    spec1 = pltpu.PrefetchScalarGridSpec(
        num_scalar_prefetch=3,
        grid=(NB,),
        in_specs=[
            pl.BlockSpec((BT, DIM), lambda b, be, xb, lv: (xb[b], 0)),
            pl.BlockSpec((1, INNER, DIM), lambda b, be, xb, lv: (be[b], 0, 0)),
            pl.BlockSpec((1, 1, INNER), lambda b, be, xb, lv: (be[b], 0, 0)),
        ],
        out_specs=pl.BlockSpec((BT, INNER), lambda b, be, xb, lv: (xb[b], 0)),
    )
    hmat = pl.pallas_call(
        _ffn1_body,
        grid_spec=spec1,
        out_shape=jax.ShapeDtypeStruct((R, INNER), jnp.bfloat16),
    )(be, xb, live, x_sorted, exp_W1, b1)
    spec2 = pltpu.PrefetchScalarGridSpec(
        num_scalar_prefetch=3,
        grid=(NB,),
        in_specs=[
            pl.BlockSpec((BT, INNER), lambda b, be, xb, lv: (xb[b], 0)),
            pl.BlockSpec((1, DIM, INNER), lambda b, be, xb, lv: (be[b], 0, 0)),
            pl.BlockSpec((1, 1, DIM), lambda b, be, xb, lv: (be[b], 0, 0)),
            pl.BlockSpec((1, 1, BT), lambda b, be, xb, lv: (xb[b], 0, 0)),
        ],
        out_specs=pl.BlockSpec((BT, DIM), lambda b, be, xb, lv: (xb[b], 0)),
    )
    return pl.pallas_call(
        _ffn2_body,
        grid_spec=spec2,
        out_shape=jax.ShapeDtypeStruct((R, DIM), jnp.float32),
    )(be, xb, live, hmat, exp_W2, b2, sw)


def _shared_body(x_ref, w1_ref, b1_ref, w2_ref, b2_ref, z0_ref, z1_ref,
                 out_ref):
    x = x_ref[...].astype(jnp.bfloat16)
    h = lax.dot_general(
        x, w1_ref[...].astype(jnp.bfloat16), (((1,), (1,)), ((), ())),
        preferred_element_type=jnp.float32)
    h = _gelu_exact(h + b1_ref[0])
    y = lax.dot_general(
        h.astype(jnp.bfloat16), w2_ref[...], (((1,), (1,)), ((), ())),
        preferred_element_type=jnp.float32)
    out_ref[...] = y + b2_ref[0] + z0_ref[...] + z1_ref[...]


def _shared_combine(x, sh_W1, sh_b1, sh_W2, sh_b2, z):
    w2 = sh_W2.astype(jnp.bfloat16)
    b1 = sh_b1.reshape(1, INNER)
    b2 = sh_b2.reshape(1, DIM)
    nblk = N_TOKENS // BT
    return pl.pallas_call(
        _shared_body,
        grid=(nblk,),
        in_specs=[
            pl.BlockSpec((BT, DIM), lambda b: (b, 0)),
            pl.BlockSpec((INNER, DIM), lambda b: (0, 0)),
            pl.BlockSpec((1, INNER), lambda b: (0, 0)),
            pl.BlockSpec((DIM, INNER), lambda b: (0, 0)),
            pl.BlockSpec((1, DIM), lambda b: (0, 0)),
            pl.BlockSpec((BT, DIM), lambda b: (b, 0)),
            pl.BlockSpec((BT, DIM), lambda b: (b + nblk, 0)),
        ],
        out_specs=pl.BlockSpec((BT, DIM), lambda b: (b, 0)),
        out_shape=jax.ShapeDtypeStruct((N_TOKENS, DIM), jnp.float32),
    )(x, sh_W1, b1, w2, b2, z, z)


def kernel(hidden_states, gate_w, exp_W1, exp_b1, exp_W2, exp_b2, sh_W1,
           sh_b1, sh_W2, sh_b2):
    b, s, d = hidden_states.shape
    x = hidden_states.reshape(-1, d)
    topk_w, topk_idx = _gate(x, gate_w)
    sorted_tok, sorted_w, sorted_dest, be, xb, live = _route_metadata(
        topk_w, topk_idx)
    x_sorted = _sc_gather(x, sorted_tok)
    y_sorted = _group_ffn(x_sorted, sorted_w, be, xb, live,
                          exp_W1, exp_b1, exp_W2, exp_b2)
    z = _sc_scatter(y_sorted, sorted_dest.reshape(NW, NCH, CHUNK))
    y = _shared_combine(x, sh_W1, sh_b1, sh_W2, sh_b2, z)
    return y.reshape(b, s, d)
